# baseline (speedup 1.0000x reference)
import functools
import jax
import jax.numpy as jnp
from jax import lax
from jax.experimental import pallas as pl
from jax.experimental.pallas import tpu as pltpu, tpu_sc as plsc


def kernel(x, pos_emb):
    B, L, D = x.shape
    info = plsc.get_sparse_core_info()
    NC, NS, LANES = info.num_cores, info.num_subcores, info.num_lanes
    NW = NC * NS
    rows_per_w = L // NW
    SUB = 32
    nsub = rows_per_w // SUB

    mesh = plsc.VectorSubcoreMesh(core_axis_name="c", subcore_axis_name="s")

    @functools.partial(
        pl.kernel,
        mesh=mesh,
        out_type=jax.ShapeDtypeStruct((B, L, D), jnp.float32),
        scratch_types=[
            pltpu.VMEM((SUB, D), jnp.float32),
            pltpu.VMEM((SUB, D), jnp.float32),
        ],
    )
    def sc_add(x_hbm, pe_hbm, out_hbm, pe_v, x_v):
        wid = lax.axis_index("s") * NC + lax.axis_index("c")
        pos0 = wid * rows_per_w
        for sub in range(nsub):
            base = pos0 + sub * SUB
            pltpu.sync_copy(pe_hbm.at[pl.ds(base, SUB), :], pe_v)
            for b in range(B):
                pltpu.sync_copy(x_hbm.at[b, pl.ds(base, SUB), :], x_v)

                def row(r, _):
                    for c in range(D // LANES):
                        sl = pl.ds(c * LANES, LANES)
                        x_v[r, sl] = x_v[r, sl] + pe_v[r, sl]
                    return 0

                lax.fori_loop(0, SUB, row, 0)
                pltpu.sync_copy(x_v, out_hbm.at[b, pl.ds(base, SUB), :])

    return sc_add(x, pos_emb)
